# baseline (speedup 1.0000x reference)
import jax
import jax.numpy as jnp
from jax.experimental import pallas as pl
from jax.experimental.pallas import tpu as pltpu

NUM_EXPERTS = 16
HIDDEN_DIM = 1024
EXPERT_DIM = 512
T = 32
TOP_K = 2


EXPERTS_PER_STEP = 2


def _moe_kernel(idx_ref, scores_ref, x_ref, up_ref, dn_ref, out_ref):
    step = pl.program_id(0)
    contrib = jnp.zeros((T, HIDDEN_DIM), jnp.float32)
    for j in range(EXPERTS_PER_STEP):
        e = step * EXPERTS_PER_STEP + j
        mask = (idx_ref[...] == e).astype(jnp.float32)
        w = jnp.sum(scores_ref[...] * mask, axis=1)

        h = jnp.dot(x_ref[...], up_ref[j], preferred_element_type=jnp.float32)
        gate = h[:, :EXPERT_DIM]
        up = h[:, EXPERT_DIM:]
        y = (gate * jax.nn.sigmoid(gate)) * up
        o = jnp.dot(y, dn_ref[j], preferred_element_type=jnp.float32)
        contrib = contrib + o * w[:, None]

    @pl.when(step == 0)
    def _init():
        out_ref[...] = contrib

    @pl.when(step != 0)
    def _acc():
        out_ref[...] += contrib


@jax.jit
def kernel(x, expert_indices, scores, up_proj, down_proj):
    grid = (NUM_EXPERTS // EXPERTS_PER_STEP,)
    return pl.pallas_call(
        _moe_kernel,
        grid=grid,
        in_specs=[
            pl.BlockSpec((T, TOP_K), lambda e: (0, 0)),
            pl.BlockSpec((T, TOP_K), lambda e: (0, 0)),
            pl.BlockSpec((T, HIDDEN_DIM), lambda e: (0, 0)),
            pl.BlockSpec((EXPERTS_PER_STEP, HIDDEN_DIM, 2 * EXPERT_DIM),
                         lambda e: (e, 0, 0)),
            pl.BlockSpec((EXPERTS_PER_STEP, EXPERT_DIM, HIDDEN_DIM),
                         lambda e: (e, 0, 0)),
        ],
        out_specs=pl.BlockSpec((T, HIDDEN_DIM), lambda e: (0, 0)),
        out_shape=jax.ShapeDtypeStruct((T, HIDDEN_DIM), jnp.float32),
        compiler_params=pltpu.CompilerParams(
            dimension_semantics=("arbitrary",),
        ),
    )(expert_indices, scores, x, up_proj, down_proj)


# speedup vs baseline: 1.0618x; 1.0618x over previous
import jax
import jax.numpy as jnp
from jax.experimental import pallas as pl
from jax.experimental.pallas import tpu as pltpu

NUM_EXPERTS = 16
HIDDEN_DIM = 1024
EXPERT_DIM = 512
T = 32
TOP_K = 2


def _moe_kernel(idx_ref, scores_ref, x_ref, gate_ref, up_ref, dn_ref, out_ref):
    e = pl.program_id(0)
    mask = (idx_ref[...] == e).astype(jnp.float32)
    w = jnp.sum(scores_ref[...] * mask, axis=1)

    g = jnp.dot(x_ref[...], gate_ref[0], preferred_element_type=jnp.float32)
    u = jnp.dot(x_ref[...], up_ref[0], preferred_element_type=jnp.float32)
    y = (g * jax.nn.sigmoid(g)) * u
    o = jnp.dot(y, dn_ref[0], preferred_element_type=jnp.float32)
    contrib = o * w[:, None]

    @pl.when(e == 0)
    def _init():
        out_ref[...] = contrib

    @pl.when(e != 0)
    def _acc():
        out_ref[...] += contrib


@jax.jit
def kernel(x, expert_indices, scores, up_proj, down_proj):
    grid = (NUM_EXPERTS,)
    return pl.pallas_call(
        _moe_kernel,
        grid=grid,
        in_specs=[
            pl.BlockSpec((T, TOP_K), lambda e: (0, 0)),
            pl.BlockSpec((T, TOP_K), lambda e: (0, 0)),
            pl.BlockSpec((T, HIDDEN_DIM), lambda e: (0, 0)),
            pl.BlockSpec((1, HIDDEN_DIM, EXPERT_DIM), lambda e: (e, 0, 0)),
            pl.BlockSpec((1, HIDDEN_DIM, EXPERT_DIM), lambda e: (e, 0, 1)),
            pl.BlockSpec((1, EXPERT_DIM, HIDDEN_DIM), lambda e: (e, 0, 0)),
        ],
        out_specs=pl.BlockSpec((T, HIDDEN_DIM), lambda e: (0, 0)),
        out_shape=jax.ShapeDtypeStruct((T, HIDDEN_DIM), jnp.float32),
        compiler_params=pltpu.CompilerParams(
            dimension_semantics=("arbitrary",),
        ),
    )(expert_indices, scores, x, up_proj, up_proj, down_proj)
